# Initial kernel scaffold; baseline (speedup 1.0000x reference)
import functools

import jax
import jax.numpy as jnp
from jax import lax
from jax.experimental import pallas as pl
from jax.experimental.pallas import tpu as pltpu
from jax.experimental.pallas import tpu_sc as plsc

N = 10000
E = 320000
DF = 128
DX = 144

NCORE = 2
NSUB = 16
NW = NCORE * NSUB
CHUNK = 128
NCHUNK = 80
E_PAD = NW * NCHUNK * CHUNK
NPAD = 10112
RPT = NPAD // NSUB

BLK = 1000


def _sc_segsum_body(xx_hbm, idxi_hbm, idxj_hbm, zeros_hbm, out_hbm,
                    idxi_v, idxj_v, buf0, buf1, acc, sem0, sem1):
    c = lax.axis_index("c")
    s = lax.axis_index("s")
    w = c * NSUB + s

    pltpu.sync_copy(zeros_hbm.at[pl.ds(s * RPT, RPT)],
                    acc.at[pl.ds(s * RPT, RPT)])
    pltpu.sync_copy(idxi_hbm.at[w], idxi_v)
    pltpu.sync_copy(idxj_hbm.at[w], idxj_v)
    plsc.subcore_barrier()

    def issue(ch, buf, sem):
        pltpu.async_copy(xx_hbm.at[idxj_v.at[ch]], buf, sem)

    def wait(ch, buf, sem):
        pltpu.make_async_copy(xx_hbm.at[idxj_v.at[ch]], buf, sem).wait()

    issue(0, buf0, sem0)
    issue(1, buf1, sem1)

    def body(p, carry):
        for b, (buf, sem) in enumerate(((buf0, sem0), (buf1, sem1))):
            ch = p * 2 + b
            wait(ch, buf, sem)
            pltpu.sync_copy(buf, acc.at[idxi_v.at[ch]], add=True)

            @pl.when(ch + 2 < NCHUNK)
            def _():
                issue(ch + 2, buf, sem)
        return carry

    lax.fori_loop(0, NCHUNK // 2, body, 0)

    plsc.subcore_barrier()
    pltpu.sync_copy(acc.at[pl.ds(s * RPT, RPT)],
                    out_hbm.at[c, pl.ds(s * RPT, RPT)])


_sc_segsum = functools.partial(
    pl.kernel,
    out_type=jax.ShapeDtypeStruct((NCORE, NPAD, DX), jnp.float32),
    mesh=plsc.VectorSubcoreMesh(core_axis_name="c", subcore_axis_name="s"),
    scratch_types=[
        pltpu.VMEM((NCHUNK, CHUNK), jnp.int32),
        pltpu.VMEM((NCHUNK, CHUNK), jnp.int32),
        pltpu.VMEM((CHUNK, DX), jnp.float32),
        pltpu.VMEM((CHUNK, DX), jnp.float32),
        pltpu.VMEM_SHARED((NPAD, DX), jnp.float32),
        pltpu.SemaphoreType.DMA,
        pltpu.SemaphoreType.DMA,
    ],
)(_sc_segsum_body)


def _tc_dense_body(x_ref, pos_ref, s_ref, w1m_ref, b1m_ref, w1a_ref, b1a_ref,
                   wxi_ref, wdx_ref, wdp_ref, wctx_ref, b2m_ref,
                   w2ax_ref, w2aa_ref, b2a_ref, o_ref):
    f32 = jnp.float32
    x = x_ref[...]
    pos = pos_ref[...]
    st = s_ref[0] + s_ref[1]
    sum_x = st[:, :DF]
    sum_pos = st[:, DF:DF + 3]
    cnt = st[:, DF + 3:DF + 4]
    inv = 1.0 / jnp.maximum(cnt, 1.0)
    nz = cnt > 0.0
    mean_x = sum_x * inv
    dpos = sum_pos * inv - pos
    aggr1 = jnp.where(
        nz, jnp.dot(dpos, w1m_ref[...], preferred_element_type=f32)
        + b1m_ref[...], 0.0)
    ctx = jnp.dot(aggr1, w1a_ref[...], preferred_element_type=f32) + b1a_ref[...]
    aggr2 = jnp.where(
        nz,
        jnp.dot(x, wxi_ref[...], preferred_element_type=f32)
        + jnp.dot(mean_x, wdx_ref[...], preferred_element_type=f32)
        + jnp.dot(dpos, wdp_ref[...], preferred_element_type=f32)
        + jnp.dot(ctx, wctx_ref[...], preferred_element_type=f32)
        + b2m_ref[...],
        0.0)
    o_ref[...] = (jnp.dot(x, w2ax_ref[...], preferred_element_type=f32)
                  + jnp.dot(aggr2, w2aa_ref[...], preferred_element_type=f32)
                  + b2a_ref[...])


def _tc_dense(x, pos, partials, *weights, interpret=False):
    def _full(w):
        return pl.BlockSpec(w.shape, lambda i: (0,) * w.ndim)

    return pl.pallas_call(
        _tc_dense_body,
        grid=(N // BLK,),
        in_specs=[
            pl.BlockSpec((BLK, DF), lambda i: (i, 0)),
            pl.BlockSpec((BLK, 3), lambda i: (i, 0)),
            pl.BlockSpec((NCORE, BLK, DX), lambda i: (0, i, 0)),
        ] + [_full(w) for w in weights],
        out_specs=pl.BlockSpec((BLK, DF), lambda i: (i, 0)),
        out_shape=jax.ShapeDtypeStruct((N, DF), jnp.float32),
        interpret=interpret,
    )(x, pos, partials, *weights)


def kernel(x, edge_index, pos, W1m, b1m, W1a, b1a, W2m, b2m, W2a, b2a):
    idx_i = edge_index[0].astype(jnp.int32)
    idx_j = edge_index[1].astype(jnp.int32)
    xx = jnp.concatenate(
        [x, pos, jnp.ones((N, 1), jnp.float32),
         jnp.zeros((N, DX - DF - 4), jnp.float32)], axis=1)
    padn = E_PAD - E
    idx_i = jnp.concatenate([idx_i, jnp.full((padn,), N, jnp.int32)])
    idx_j = jnp.concatenate([idx_j, jnp.zeros((padn,), jnp.int32)])
    idx_i = idx_i.reshape(NW, NCHUNK, CHUNK)
    idx_j = idx_j.reshape(NW, NCHUNK, CHUNK)
    zeros = jnp.zeros((NPAD, DX), jnp.float32)

    partials = _sc_segsum(xx, idx_i, idx_j, zeros)

    return _tc_dense(
        x, pos, partials,
        W1m, b1m.reshape(1, -1), W1a, b1a.reshape(1, -1),
        W2m[0:DF] - W2m[DF:2 * DF], W2m[DF:2 * DF],
        W2m[2 * DF:2 * DF + 3], W2m[2 * DF + 3:],
        b2m.reshape(1, -1), W2a[:DF], W2a[DF:], b2a.reshape(1, -1))


# speedup vs baseline: 12.0911x; 12.0911x over previous
import functools

import jax
import jax.numpy as jnp
from jax import lax
from jax.experimental import pallas as pl
from jax.experimental.pallas import tpu as pltpu
from jax.experimental.pallas import tpu_sc as plsc

N = 10000
E = 320000
DF = 128
DH = 64
DP = 16

NCORE = 2
NSUB = 16
CHUNK = 128
NCHUNK = 160
PCHUNK = 80
E_PAD = NSUB * NCHUNK * CHUNK
NPAD = 10112
RPT = NPAD // NSUB
XB = 4
PB = 2

BLK = 1000


def _sc_segsum_body(xlo_hbm, xhi_hbm, pp_hbm, idxi_hbm, idxj_hbm,
                    zx_hbm, zp_hbm, outx_hbm, outp_hbm,
                    idxi_v, idxj_v, xbufs, pbufs, accx, accp, xsems, psems):
    c = lax.axis_index("c")
    s = lax.axis_index("s")

    rows = pl.ds(s * RPT, RPT)
    pltpu.sync_copy(zx_hbm.at[rows], accx.at[rows])
    pltpu.sync_copy(zp_hbm.at[rows], accp.at[rows])
    pltpu.sync_copy(idxi_hbm.at[s], idxi_v)
    pltpu.sync_copy(idxj_hbm.at[s], idxj_v)
    plsc.subcore_barrier()

    pbase = c * PCHUNK

    def x_issue(ch, b):
        @pl.when(c == 0)
        def _():
            pltpu.async_copy(xlo_hbm.at[idxj_v.at[ch]], xbufs[b], xsems[b])

        @pl.when(c != 0)
        def _():
            pltpu.async_copy(xhi_hbm.at[idxj_v.at[ch]], xbufs[b], xsems[b])

    def x_wait(b):
        pltpu.make_async_copy(xlo_hbm.at[idxj_v.at[0]], xbufs[b],
                              xsems[b]).wait()

    def p_issue(ch, b):
        pltpu.async_copy(pp_hbm.at[idxj_v.at[ch]], pbufs[b], psems[b])

    def p_wait(b):
        pltpu.make_async_copy(pp_hbm.at[idxj_v.at[0]], pbufs[b],
                              psems[b]).wait()

    for b in range(XB):
        x_issue(b, b)
    for b in range(PB):
        p_issue(pbase + b, b)

    def body(p, carry):
        for b in range(XB):
            ch = p * XB + b
            x_wait(b)
            pltpu.sync_copy(xbufs[b], accx.at[idxi_v.at[ch]], add=True)

            @pl.when(ch + XB < NCHUNK)
            def _():
                x_issue(ch + XB, b)

        for b in range(PB):
            q = p * PB + b
            p_wait(b)
            pltpu.sync_copy(pbufs[b], accp.at[idxi_v.at[pbase + q]], add=True)

            @pl.when(q + PB < PCHUNK)
            def _():
                p_issue(pbase + q + PB, b)

        return carry

    lax.fori_loop(0, NCHUNK // XB, body, 0)

    plsc.subcore_barrier()
    pltpu.sync_copy(accx.at[rows], outx_hbm.at[c, rows])
    pltpu.sync_copy(accp.at[rows], outp_hbm.at[c, rows])


@functools.cache
def _sc_segsum():
    return pl.kernel(
        _sc_segsum_body,
        out_type=(
            jax.ShapeDtypeStruct((NCORE, NPAD, DH), jnp.float32),
            jax.ShapeDtypeStruct((NCORE, NPAD, DP), jnp.float32),
        ),
        mesh=plsc.VectorSubcoreMesh(core_axis_name="c", subcore_axis_name="s"),
        compiler_params=pltpu.CompilerParams(use_tc_tiling_on_sc=False),
        scratch_types=[
            pltpu.VMEM((NCHUNK, CHUNK), jnp.int32),
            pltpu.VMEM((NCHUNK, CHUNK), jnp.int32),
            [pltpu.VMEM((CHUNK, DH), jnp.float32) for _ in range(XB)],
            [pltpu.VMEM((CHUNK, DP), jnp.float32) for _ in range(PB)],
            pltpu.VMEM_SHARED((NPAD, DH), jnp.float32),
            pltpu.VMEM_SHARED((NPAD, DP), jnp.float32),
            [pltpu.SemaphoreType.DMA for _ in range(XB)],
            [pltpu.SemaphoreType.DMA for _ in range(PB)],
        ],
    )


def _tc_dense_body(x_ref, pos_ref, sx_ref, sp_ref,
                   w1m_ref, b1m_ref, w1a_ref, b1a_ref,
                   wxi_ref, wdx_ref, wdp_ref, wctx_ref, b2m_ref,
                   w2ax_ref, w2aa_ref, b2a_ref, o_ref):
    f32 = jnp.float32
    x = x_ref[...]
    pos = pos_ref[...]
    sum_x = jnp.concatenate([sx_ref[0], sx_ref[1]], axis=1)
    sp = sp_ref[0] + sp_ref[1]
    sum_pos = sp[:, :3]
    cnt = sp[:, 3:4]
    inv = 1.0 / jnp.maximum(cnt, 1.0)
    nz = cnt > 0.0
    mean_x = sum_x * inv
    dpos = sum_pos * inv - pos
    aggr1 = jnp.where(
        nz, jnp.dot(dpos, w1m_ref[...], preferred_element_type=f32)
        + b1m_ref[...], 0.0)
    ctx = jnp.dot(aggr1, w1a_ref[...], preferred_element_type=f32) + b1a_ref[...]
    aggr2 = jnp.where(
        nz,
        jnp.dot(x, wxi_ref[...], preferred_element_type=f32)
        + jnp.dot(mean_x, wdx_ref[...], preferred_element_type=f32)
        + jnp.dot(dpos, wdp_ref[...], preferred_element_type=f32)
        + jnp.dot(ctx, wctx_ref[...], preferred_element_type=f32)
        + b2m_ref[...],
        0.0)
    o_ref[...] = (jnp.dot(x, w2ax_ref[...], preferred_element_type=f32)
                  + jnp.dot(aggr2, w2aa_ref[...], preferred_element_type=f32)
                  + b2a_ref[...])


def _tc_dense(x, pos, part_x, part_p, *weights, interpret=False):
    def _full(w):
        return pl.BlockSpec(w.shape, lambda i: (0,) * w.ndim)

    return pl.pallas_call(
        _tc_dense_body,
        grid=(N // BLK,),
        in_specs=[
            pl.BlockSpec((BLK, DF), lambda i: (i, 0)),
            pl.BlockSpec((BLK, 3), lambda i: (i, 0)),
            pl.BlockSpec((NCORE, BLK, DH), lambda i: (0, i, 0)),
            pl.BlockSpec((NCORE, BLK, DP), lambda i: (0, i, 0)),
        ] + [_full(w) for w in weights],
        out_specs=pl.BlockSpec((BLK, DF), lambda i: (i, 0)),
        out_shape=jax.ShapeDtypeStruct((N, DF), jnp.float32),
        interpret=interpret,
    )(x, pos, part_x, part_p, *weights)


def kernel(x, edge_index, pos, W1m, b1m, W1a, b1a, W2m, b2m, W2a, b2a):
    idx_i = edge_index[0].astype(jnp.int32)
    idx_j = edge_index[1].astype(jnp.int32)
    pp = jnp.concatenate(
        [pos, jnp.ones((N, 1), jnp.float32),
         jnp.zeros((N, DP - 4), jnp.float32)], axis=1)
    padn = E_PAD - E
    idx_i = jnp.concatenate([idx_i, jnp.full((padn,), N, jnp.int32)])
    idx_j = jnp.concatenate([idx_j, jnp.zeros((padn,), jnp.int32)])
    idx_i = idx_i.reshape(NSUB, NCHUNK, CHUNK)
    idx_j = idx_j.reshape(NSUB, NCHUNK, CHUNK)
    zx = jnp.zeros((NPAD, DH), jnp.float32)
    zp = jnp.zeros((NPAD, DP), jnp.float32)

    part_x, part_p = _sc_segsum()(
        x[:, :DH], x[:, DH:], pp, idx_i, idx_j, zx, zp)

    return _tc_dense(
        x, pos, part_x, part_p,
        W1m, b1m.reshape(1, -1), W1a, b1a.reshape(1, -1),
        W2m[0:DF] - W2m[DF:2 * DF], W2m[DF:2 * DF],
        W2m[2 * DF:2 * DF + 3], W2m[2 * DF + 3:],
        b2m.reshape(1, -1), W2a[:DF], W2a[DF:], b2a.reshape(1, -1))
